# baseline (speedup 1.0000x reference)
import functools

import jax
import jax.numpy as jnp
from jax import lax
from jax.experimental import pallas as pl
from jax.experimental.pallas import tpu as pltpu
from jax.experimental.pallas import tpu_sc as plsc

N = 10000
F = 128
EMB = 32
H = 128
G = 10
E = 320000

NBLK = 10
BLK = N // NBLK

NC = 2
NS = 16
NW = NC * NS
CHUNK = 100
EPW = E // NW
NCHUNK = EPW // CHUNK
NPAIR = NCHUNK // 2
RPT = N // NS


def _embed_mm_body(f_ref, emb_ref, w1a_ref, w1p_ref, y_ref):
    f = f_ref[...]
    op = f[:, 0:1].astype(jnp.int32)
    cols = lax.broadcasted_iota(jnp.int32, (BLK, 128), 1)
    onehot = (op == cols).astype(jnp.float32)
    emb = jnp.dot(onehot, emb_ref[...], preferred_element_type=jnp.float32)
    y = jnp.dot(emb, w1a_ref[...], preferred_element_type=jnp.float32)
    y += jnp.dot(f, w1p_ref[...], preferred_element_type=jnp.float32)
    y_ref[...] = y


def _embed_mm(f, emb_table, w1a, w1p):
    return pl.pallas_call(
        _embed_mm_body,
        grid=(NBLK,),
        in_specs=[
            pl.BlockSpec((BLK, F), lambda i: (i, 0)),
            pl.BlockSpec((128, EMB), lambda i: (0, 0)),
            pl.BlockSpec((EMB, H), lambda i: (0, 0)),
            pl.BlockSpec((F, H), lambda i: (0, 0)),
        ],
        out_specs=pl.BlockSpec((BLK, H), lambda i: (i, 0)),
        out_shape=jax.ShapeDtypeStruct((N, H), jnp.float32),
    )(f, emb_table, w1a, w1p)


@functools.cache
def _make_edge_pass():
    mesh = plsc.VectorSubcoreMesh(
        core_axis_name="c", subcore_axis_name="s",
        num_cores=NC, num_subcores=NS)

    @functools.partial(
        pl.kernel,
        out_type=jax.ShapeDtypeStruct((NC, NS, RPT, H), jnp.float32),
        mesh=mesh,
        scratch_types=[
            pltpu.VMEM_SHARED((N, H), jnp.float32),
            pltpu.VMEM((NCHUNK, CHUNK), jnp.int32),
            pltpu.VMEM((1, CHUNK), jnp.int32),
            pltpu.VMEM((1, CHUNK), jnp.int32),
            pltpu.VMEM((CHUNK, H), jnp.float32),
            pltpu.VMEM((CHUNK, H), jnp.float32),
            pltpu.SemaphoreType.DMA,
            pltpu.SemaphoreType.DMA,
            pltpu.SemaphoreType.DMA,
            pltpu.SemaphoreType.DMA,
            pltpu.SemaphoreType.DMA,
            pltpu.SemaphoreType.DMA,
        ],
    )
    def edge_pass(y_hbm, src_hbm, dst_hbm, zeros_hbm, out_hbm,
                  acc, dst_v, srcb0, srcb1, rows0, rows1,
                  sem0, sem1, sem_s0, sem_s1, sem_t0, sem_t1):
        cid = lax.axis_index("c")
        sid = lax.axis_index("s")
        wid = cid * NS + sid

        pltpu.sync_copy(dst_hbm.at[wid], dst_v)
        pltpu.async_copy(src_hbm.at[wid, 0], srcb0, sem_s0)
        pltpu.async_copy(src_hbm.at[wid, 1], srcb1, sem_s1)
        pltpu.sync_copy(zeros_hbm.at[sid], acc.at[pl.ds(sid * RPT, RPT)])
        plsc.subcore_barrier()

        pltpu.async_copy(y_hbm.at[dst_v.at[0]], rows0, sem0)
        pltpu.async_copy(y_hbm.at[dst_v.at[1]], rows1, sem1)

        def body(p, carry):
            c0 = 2 * p
            pltpu.make_async_copy(y_hbm.at[dst_v.at[c0]], rows0, sem0).wait()
            pltpu.make_async_copy(src_hbm.at[wid, c0], srcb0, sem_s0).wait()
            pltpu.async_copy(rows0, acc.at[srcb0.at[0]], sem_t0, add=True)
            pltpu.make_async_copy(
                y_hbm.at[dst_v.at[c0 + 1]], rows1, sem1).wait()
            pltpu.make_async_copy(
                src_hbm.at[wid, c0 + 1], srcb1, sem_s1).wait()
            pltpu.async_copy(rows1, acc.at[srcb1.at[0]], sem_t1, add=True)
            pltpu.make_async_copy(
                rows0, acc.at[srcb0.at[0]], sem_t0).wait()
            pltpu.async_copy(src_hbm.at[wid, c0 + 2], srcb0, sem_s0)
            pltpu.async_copy(y_hbm.at[dst_v.at[c0 + 2]], rows0, sem0)
            pltpu.make_async_copy(
                rows1, acc.at[srcb1.at[0]], sem_t1).wait()
            pltpu.async_copy(src_hbm.at[wid, c0 + 3], srcb1, sem_s1)
            pltpu.async_copy(y_hbm.at[dst_v.at[c0 + 3]], rows1, sem1)
            return carry

        lax.fori_loop(0, NPAIR - 1, body, 0)
        c0 = NCHUNK - 2
        pltpu.make_async_copy(y_hbm.at[dst_v.at[c0]], rows0, sem0).wait()
        pltpu.make_async_copy(src_hbm.at[wid, c0], srcb0, sem_s0).wait()
        pltpu.async_copy(rows0, acc.at[srcb0.at[0]], sem_t0, add=True)
        pltpu.make_async_copy(y_hbm.at[dst_v.at[c0 + 1]], rows1, sem1).wait()
        pltpu.make_async_copy(src_hbm.at[wid, c0 + 1], srcb1, sem_s1).wait()
        pltpu.async_copy(rows1, acc.at[srcb1.at[0]], sem_t1, add=True)
        pltpu.make_async_copy(rows0, acc.at[srcb0.at[0]], sem_t0).wait()
        pltpu.make_async_copy(rows1, acc.at[srcb1.at[0]], sem_t1).wait()
        plsc.subcore_barrier()

        pltpu.sync_copy(acc.at[pl.ds(sid * RPT, RPT)], out_hbm.at[cid, sid])

    return edge_pass


@functools.cache
def _make_deg_pass():
    mesh = plsc.VectorSubcoreMesh(
        core_axis_name="c", subcore_axis_name="s",
        num_cores=NC, num_subcores=NS)

    @functools.partial(
        pl.kernel,
        out_type=jax.ShapeDtypeStruct((NC, NS, RPT, H), jnp.float32),
        mesh=mesh,
        scratch_types=[
            pltpu.VMEM_SHARED((N, H), jnp.float32),
            pltpu.VMEM((NCHUNK, CHUNK), jnp.int32),
            pltpu.VMEM((CHUNK, H), jnp.float32),
        ],
    )
    def deg_pass(src_hbm, zeros_hbm, ones_hbm, deg_hbm,
                 degacc, src_v, ones_v):
        cid = lax.axis_index("c")
        sid = lax.axis_index("s")
        wid = cid * NS + sid

        pltpu.sync_copy(src_hbm.at[wid], src_v)
        pltpu.sync_copy(zeros_hbm.at[sid], degacc.at[pl.ds(sid * RPT, RPT)])
        pltpu.sync_copy(ones_hbm, ones_v)
        plsc.subcore_barrier()

        def body(c, carry):
            pltpu.sync_copy(ones_v, degacc.at[src_v.at[c]], add=True)
            return carry

        lax.fori_loop(0, NCHUNK, body, 0)
        plsc.subcore_barrier()
        pltpu.sync_copy(degacc.at[pl.ds(sid * RPT, RPT)], deg_hbm.at[cid, sid])

    return deg_pass


def _edge_pass(*args):
    return _make_edge_pass()(*args)


def _deg_pass(*args):
    return _make_deg_pass()(*args)


def _relu_mm_body(p_ref, d_ref, b_ref, w_ref, y_ref):
    m = p_ref[0] + p_ref[1]
    deg = (d_ref[0] + d_ref[1])[:, 0:1]
    dinv = 1.0 / jnp.maximum(deg, 1.0)
    h = jnp.maximum(m * dinv + b_ref[...], 0.0)
    y_ref[...] = jnp.dot(h, w_ref[...], preferred_element_type=jnp.float32)


def _relu_mm(p, d, b, w):
    return pl.pallas_call(
        _relu_mm_body,
        grid=(NBLK,),
        in_specs=[
            pl.BlockSpec((NC, BLK, H), lambda i: (0, i, 0)),
            pl.BlockSpec((NC, BLK, H), lambda i: (0, i, 0)),
            pl.BlockSpec((1, H), lambda i: (0, 0)),
            pl.BlockSpec((H, H), lambda i: (0, 0)),
        ],
        out_specs=pl.BlockSpec((BLK, H), lambda i: (i, 0)),
        out_shape=jax.ShapeDtypeStruct((N, H), jnp.float32),
    )(p, d, b, w)


def _final_body(p_ref, d_ref, b_ref, wp_ref, bp_ref, o_ref):
    m = p_ref[0] + p_ref[1]
    deg = (d_ref[0] + d_ref[1])[:, 0:1]
    dinv = 1.0 / jnp.maximum(deg, 1.0)
    h = jnp.maximum(m * dinv + b_ref[...], 0.0)
    z = jnp.sum(h * wp_ref[...], axis=1, keepdims=True)
    r = jnp.abs(z + bp_ref[0:1, 0:1])
    o_ref[...] = jnp.broadcast_to(jnp.sum(r), (1, 1, 128))


def _final(p, d, b, wp, bp):
    return pl.pallas_call(
        _final_body,
        grid=(NBLK,),
        in_specs=[
            pl.BlockSpec((NC, BLK, H), lambda i: (0, i, 0)),
            pl.BlockSpec((NC, BLK, H), lambda i: (0, i, 0)),
            pl.BlockSpec((1, H), lambda i: (0, 0)),
            pl.BlockSpec((1, H), lambda i: (0, 0)),
            pl.BlockSpec((1, 128), lambda i: (0, 0)),
        ],
        out_specs=pl.BlockSpec((1, 1, 128), lambda i: (i, 0, 0)),
        out_shape=jax.ShapeDtypeStruct((G, 1, 128), jnp.float32),
    )(p, d, b, wp, bp)


def kernel(features, edge_index, lengths, emb_table, W1, b1, W2, b2, Wp, bp):
    del lengths
    f = features[0]
    src3d = edge_index[0].reshape(NW, NCHUNK, CHUNK)
    src4d = edge_index[0].reshape(NW, NCHUNK, 1, CHUNK)
    dst3d = edge_index[1].reshape(NW, NCHUNK, CHUNK)
    w1a = W1[:EMB]
    w1p = jnp.concatenate([jnp.zeros((1, H), W1.dtype), W1[EMB:]], axis=0)
    b1r = b1.reshape(1, H)
    b2r = b2.reshape(1, H)
    wpr = Wp.reshape(1, H)
    bpr = jnp.broadcast_to(bp.reshape(1, 1), (1, 128))
    zeros_init = jnp.zeros((NS, RPT, H), jnp.float32)
    ones_rows = jnp.ones((CHUNK, H), jnp.float32)

    y1 = _embed_mm(f, emb_table, w1a, w1p)
    d1 = _deg_pass(src3d, zeros_init, ones_rows).reshape(NC, N, H)
    p1 = _edge_pass(y1, src4d, dst3d, zeros_init).reshape(NC, N, H)
    y2 = _relu_mm(p1, d1, b1r, W2)
    p2 = _edge_pass(y2, src4d, dst3d, zeros_init).reshape(NC, N, H)
    out = _final(p2, d1, b2r, wpr, bpr)
    return out[:, 0, 0]


# speedup vs baseline: 1.1887x; 1.1887x over previous
import functools

import jax
import jax.numpy as jnp
from jax import lax
from jax.experimental import pallas as pl
from jax.experimental.pallas import tpu as pltpu
from jax.experimental.pallas import tpu_sc as plsc

N = 10000
F = 128
EMB = 32
H = 128
G = 10
E = 320000

NBLK = 10
BLK = N // NBLK

NC = 2
NS = 16
NW = NC * NS
CHUNK = 100
EPW = E // NW
NCHUNK = EPW // CHUNK
NPAIR = NCHUNK // 2
RPT = N // NS


def _embed_mm_body(f_ref, emb_ref, w1a_ref, w1p_ref, y_ref):
    f = f_ref[...]
    op = f[:, 0:1].astype(jnp.int32)
    cols = lax.broadcasted_iota(jnp.int32, (BLK, 128), 1)
    onehot = (op == cols).astype(jnp.float32)
    emb = jnp.dot(onehot, emb_ref[...], preferred_element_type=jnp.float32)
    y = jnp.dot(emb, w1a_ref[...], preferred_element_type=jnp.float32)
    y += jnp.dot(f, w1p_ref[...], preferred_element_type=jnp.float32)
    y_ref[...] = y


def _embed_mm(f, emb_table, w1a, w1p):
    return pl.pallas_call(
        _embed_mm_body,
        grid=(NBLK,),
        in_specs=[
            pl.BlockSpec((BLK, F), lambda i: (i, 0)),
            pl.BlockSpec((128, EMB), lambda i: (0, 0)),
            pl.BlockSpec((EMB, H), lambda i: (0, 0)),
            pl.BlockSpec((F, H), lambda i: (0, 0)),
        ],
        out_specs=pl.BlockSpec((BLK, H), lambda i: (i, 0)),
        out_shape=jax.ShapeDtypeStruct((N, H), jnp.float32),
    )(f, emb_table, w1a, w1p)


@functools.cache
def _make_edge_pass():
    mesh = plsc.VectorSubcoreMesh(
        core_axis_name="c", subcore_axis_name="s",
        num_cores=NC, num_subcores=NS)

    @functools.partial(
        pl.kernel,
        out_type=jax.ShapeDtypeStruct((NC, NS, RPT, H), jnp.float32),
        mesh=mesh,
        scratch_types=[
            pltpu.VMEM_SHARED((N, H), jnp.float32),
            pltpu.VMEM((NCHUNK, CHUNK), jnp.int32),
            pltpu.VMEM((1, CHUNK), jnp.int32),
            pltpu.VMEM((1, CHUNK), jnp.int32),
            pltpu.VMEM((CHUNK, H), jnp.float32),
            pltpu.VMEM((CHUNK, H), jnp.float32),
            pltpu.SemaphoreType.DMA,
            pltpu.SemaphoreType.DMA,
            pltpu.SemaphoreType.DMA,
            pltpu.SemaphoreType.DMA,
            pltpu.SemaphoreType.DMA,
            pltpu.SemaphoreType.DMA,
        ],
    )
    def edge_pass(y_hbm, src_hbm, dst_hbm, zeros_hbm, out_hbm,
                  acc, dst_v, srcb0, srcb1, rows0, rows1,
                  sem0, sem1, sem_s0, sem_s1, sem_t0, sem_t1):
        cid = lax.axis_index("c")
        sid = lax.axis_index("s")
        wid = cid * NS + sid

        pltpu.sync_copy(dst_hbm.at[wid], dst_v)
        pltpu.async_copy(src_hbm.at[wid, 0], srcb0, sem_s0)
        pltpu.async_copy(src_hbm.at[wid, 1], srcb1, sem_s1)
        pltpu.sync_copy(zeros_hbm.at[sid], acc.at[pl.ds(sid * RPT, RPT)])
        plsc.subcore_barrier()

        pltpu.async_copy(y_hbm.at[dst_v.at[0]], rows0, sem0)

        def body(p, carry):
            c0 = 2 * p
            pltpu.async_copy(y_hbm.at[dst_v.at[c0 + 1]], rows1, sem1)
            pltpu.make_async_copy(y_hbm.at[dst_v.at[c0]], rows0, sem0).wait()
            pltpu.make_async_copy(src_hbm.at[wid, c0], srcb0, sem_s0).wait()
            pltpu.sync_copy(rows0, acc.at[srcb0.at[0]], add=True)
            pltpu.async_copy(src_hbm.at[wid, c0 + 2], srcb0, sem_s0)
            pltpu.async_copy(y_hbm.at[dst_v.at[c0 + 2]], rows0, sem0)
            pltpu.make_async_copy(
                y_hbm.at[dst_v.at[c0 + 1]], rows1, sem1).wait()
            pltpu.make_async_copy(
                src_hbm.at[wid, c0 + 1], srcb1, sem_s1).wait()
            pltpu.sync_copy(rows1, acc.at[srcb1.at[0]], add=True)
            pltpu.async_copy(src_hbm.at[wid, c0 + 3], srcb1, sem_s1)
            return carry

        lax.fori_loop(0, NPAIR - 1, body, 0)
        c0 = NCHUNK - 2
        pltpu.async_copy(y_hbm.at[dst_v.at[c0 + 1]], rows1, sem1)
        pltpu.make_async_copy(y_hbm.at[dst_v.at[c0]], rows0, sem0).wait()
        pltpu.make_async_copy(src_hbm.at[wid, c0], srcb0, sem_s0).wait()
        pltpu.sync_copy(rows0, acc.at[srcb0.at[0]], add=True)
        pltpu.make_async_copy(y_hbm.at[dst_v.at[c0 + 1]], rows1, sem1).wait()
        pltpu.make_async_copy(src_hbm.at[wid, c0 + 1], srcb1, sem_s1).wait()
        pltpu.sync_copy(rows1, acc.at[srcb1.at[0]], add=True)
        plsc.subcore_barrier()

        pltpu.sync_copy(acc.at[pl.ds(sid * RPT, RPT)], out_hbm.at[cid, sid])

    return edge_pass


@functools.cache
def _make_deg_pass():
    mesh = plsc.VectorSubcoreMesh(
        core_axis_name="c", subcore_axis_name="s",
        num_cores=NC, num_subcores=NS)

    @functools.partial(
        pl.kernel,
        out_type=jax.ShapeDtypeStruct((NC, NS, RPT, H), jnp.float32),
        mesh=mesh,
        scratch_types=[
            pltpu.VMEM_SHARED((N, H), jnp.float32),
            pltpu.VMEM((NCHUNK, CHUNK), jnp.int32),
            pltpu.VMEM((CHUNK, H), jnp.float32),
        ],
    )
    def deg_pass(src_hbm, zeros_hbm, ones_hbm, deg_hbm,
                 degacc, src_v, ones_v):
        cid = lax.axis_index("c")
        sid = lax.axis_index("s")
        wid = cid * NS + sid

        pltpu.sync_copy(src_hbm.at[wid], src_v)
        pltpu.sync_copy(zeros_hbm.at[sid], degacc.at[pl.ds(sid * RPT, RPT)])
        pltpu.sync_copy(ones_hbm, ones_v)
        plsc.subcore_barrier()

        def body(c, carry):
            pltpu.sync_copy(ones_v, degacc.at[src_v.at[c]], add=True)
            return carry

        lax.fori_loop(0, NCHUNK, body, 0)
        plsc.subcore_barrier()
        pltpu.sync_copy(degacc.at[pl.ds(sid * RPT, RPT)], deg_hbm.at[cid, sid])

    return deg_pass


def _edge_pass(*args):
    return _make_edge_pass()(*args)


def _deg_pass(*args):
    return _make_deg_pass()(*args)


def _relu_mm_body(p_ref, d_ref, b_ref, w_ref, y_ref):
    m = p_ref[0] + p_ref[1]
    deg = (d_ref[0] + d_ref[1])[:, 0:1]
    dinv = 1.0 / jnp.maximum(deg, 1.0)
    h = jnp.maximum(m * dinv + b_ref[...], 0.0)
    y_ref[...] = jnp.dot(h, w_ref[...], preferred_element_type=jnp.float32)


def _relu_mm(p, d, b, w):
    return pl.pallas_call(
        _relu_mm_body,
        grid=(NBLK,),
        in_specs=[
            pl.BlockSpec((NC, BLK, H), lambda i: (0, i, 0)),
            pl.BlockSpec((NC, BLK, H), lambda i: (0, i, 0)),
            pl.BlockSpec((1, H), lambda i: (0, 0)),
            pl.BlockSpec((H, H), lambda i: (0, 0)),
        ],
        out_specs=pl.BlockSpec((BLK, H), lambda i: (i, 0)),
        out_shape=jax.ShapeDtypeStruct((N, H), jnp.float32),
    )(p, d, b, w)


def _final_body(p_ref, d_ref, b_ref, wp_ref, bp_ref, o_ref):
    m = p_ref[0] + p_ref[1]
    deg = (d_ref[0] + d_ref[1])[:, 0:1]
    dinv = 1.0 / jnp.maximum(deg, 1.0)
    h = jnp.maximum(m * dinv + b_ref[...], 0.0)
    z = jnp.sum(h * wp_ref[...], axis=1, keepdims=True)
    r = jnp.abs(z + bp_ref[0:1, 0:1])
    o_ref[...] = jnp.broadcast_to(jnp.sum(r), (1, 1, 128))


def _final(p, d, b, wp, bp):
    return pl.pallas_call(
        _final_body,
        grid=(NBLK,),
        in_specs=[
            pl.BlockSpec((NC, BLK, H), lambda i: (0, i, 0)),
            pl.BlockSpec((NC, BLK, H), lambda i: (0, i, 0)),
            pl.BlockSpec((1, H), lambda i: (0, 0)),
            pl.BlockSpec((1, H), lambda i: (0, 0)),
            pl.BlockSpec((1, 128), lambda i: (0, 0)),
        ],
        out_specs=pl.BlockSpec((1, 1, 128), lambda i: (i, 0, 0)),
        out_shape=jax.ShapeDtypeStruct((G, 1, 128), jnp.float32),
    )(p, d, b, wp, bp)


def kernel(features, edge_index, lengths, emb_table, W1, b1, W2, b2, Wp, bp):
    del lengths
    f = features[0]
    src3d = edge_index[0].reshape(NW, NCHUNK, CHUNK)
    src4d = edge_index[0].reshape(NW, NCHUNK, 1, CHUNK)
    dst3d = edge_index[1].reshape(NW, NCHUNK, CHUNK)
    w1a = W1[:EMB]
    w1p = jnp.concatenate([jnp.zeros((1, H), W1.dtype), W1[EMB:]], axis=0)
    b1r = b1.reshape(1, H)
    b2r = b2.reshape(1, H)
    wpr = Wp.reshape(1, H)
    bpr = jnp.broadcast_to(bp.reshape(1, 1), (1, 128))
    zeros_init = jnp.zeros((NS, RPT, H), jnp.float32)
    ones_rows = jnp.ones((CHUNK, H), jnp.float32)

    y1 = _embed_mm(f, emb_table, w1a, w1p)
    d1 = _deg_pass(src3d, zeros_init, ones_rows).reshape(NC, N, H)
    p1 = _edge_pass(y1, src4d, dst3d, zeros_init).reshape(NC, N, H)
    y2 = _relu_mm(p1, d1, b1r, W2)
    p2 = _edge_pass(y2, src4d, dst3d, zeros_init).reshape(NC, N, H)
    out = _final(p2, d1, b2r, wpr, bpr)
    return out[:, 0, 0]


# speedup vs baseline: 1.2232x; 1.0290x over previous
import functools

import jax
import jax.numpy as jnp
from jax import lax
from jax.experimental import pallas as pl
from jax.experimental.pallas import tpu as pltpu
from jax.experimental.pallas import tpu_sc as plsc

N = 10000
F = 128
EMB = 32
H = 128
G = 10
E = 320000

NBLK = 10
BLK = N // NBLK

NC = 2
NS = 16
NW = NC * NS
CHUNK = 125
EPW = E // NW
NCHUNK = EPW // CHUNK
NPAIR = NCHUNK // 2
RPT = N // NS


def _embed_mm_body(f_ref, emb_ref, w1a_ref, w1p_ref, y_ref):
    f = f_ref[...]
    op = f[:, 0:1].astype(jnp.int32)
    cols = lax.broadcasted_iota(jnp.int32, (BLK, 128), 1)
    onehot = (op == cols).astype(jnp.float32)
    emb = jnp.dot(onehot, emb_ref[...], preferred_element_type=jnp.float32)
    y = jnp.dot(emb, w1a_ref[...], preferred_element_type=jnp.float32)
    y += jnp.dot(f, w1p_ref[...], preferred_element_type=jnp.float32)
    y_ref[...] = y


def _embed_mm(f, emb_table, w1a, w1p):
    return pl.pallas_call(
        _embed_mm_body,
        grid=(NBLK,),
        in_specs=[
            pl.BlockSpec((BLK, F), lambda i: (i, 0)),
            pl.BlockSpec((128, EMB), lambda i: (0, 0)),
            pl.BlockSpec((EMB, H), lambda i: (0, 0)),
            pl.BlockSpec((F, H), lambda i: (0, 0)),
        ],
        out_specs=pl.BlockSpec((BLK, H), lambda i: (i, 0)),
        out_shape=jax.ShapeDtypeStruct((N, H), jnp.float32),
    )(f, emb_table, w1a, w1p)


@functools.cache
def _make_edge_pass():
    mesh = plsc.VectorSubcoreMesh(
        core_axis_name="c", subcore_axis_name="s",
        num_cores=NC, num_subcores=NS)

    @functools.partial(
        pl.kernel,
        out_type=jax.ShapeDtypeStruct((NC, NS, RPT, H), jnp.float32),
        mesh=mesh,
        scratch_types=[
            pltpu.VMEM_SHARED((N, H), jnp.float32),
            pltpu.VMEM((NCHUNK, CHUNK), jnp.int32),
            pltpu.VMEM((1, CHUNK), jnp.int32),
            pltpu.VMEM((1, CHUNK), jnp.int32),
            pltpu.VMEM((CHUNK, H), jnp.float32),
            pltpu.VMEM((CHUNK, H), jnp.float32),
            pltpu.SemaphoreType.DMA,
            pltpu.SemaphoreType.DMA,
            pltpu.SemaphoreType.DMA,
            pltpu.SemaphoreType.DMA,
            pltpu.SemaphoreType.DMA,
            pltpu.SemaphoreType.DMA,
        ],
    )
    def edge_pass(y_hbm, src_hbm, dst_hbm, zeros_hbm, out_hbm,
                  acc, dst_v, srcb0, srcb1, rows0, rows1,
                  sem0, sem1, sem_s0, sem_s1, sem_t0, sem_t1):
        cid = lax.axis_index("c")
        sid = lax.axis_index("s")
        wid = cid * NS + sid

        pltpu.sync_copy(dst_hbm.at[wid], dst_v)
        pltpu.async_copy(src_hbm.at[wid, 0], srcb0, sem_s0)
        pltpu.async_copy(src_hbm.at[wid, 1], srcb1, sem_s1)
        pltpu.sync_copy(zeros_hbm.at[sid], acc.at[pl.ds(sid * RPT, RPT)])
        plsc.subcore_barrier()

        pltpu.async_copy(y_hbm.at[dst_v.at[0]], rows0, sem0)

        def body(p, carry):
            c0 = 2 * p
            pltpu.async_copy(y_hbm.at[dst_v.at[c0 + 1]], rows1, sem1)
            pltpu.make_async_copy(y_hbm.at[dst_v.at[c0]], rows0, sem0).wait()
            pltpu.make_async_copy(src_hbm.at[wid, c0], srcb0, sem_s0).wait()
            pltpu.sync_copy(rows0, acc.at[srcb0.at[0]], add=True)
            pltpu.async_copy(src_hbm.at[wid, c0 + 2], srcb0, sem_s0)
            pltpu.async_copy(y_hbm.at[dst_v.at[c0 + 2]], rows0, sem0)
            pltpu.make_async_copy(
                y_hbm.at[dst_v.at[c0 + 1]], rows1, sem1).wait()
            pltpu.make_async_copy(
                src_hbm.at[wid, c0 + 1], srcb1, sem_s1).wait()
            pltpu.sync_copy(rows1, acc.at[srcb1.at[0]], add=True)
            pltpu.async_copy(src_hbm.at[wid, c0 + 3], srcb1, sem_s1)
            return carry

        lax.fori_loop(0, NPAIR - 1, body, 0)
        c0 = NCHUNK - 2
        pltpu.async_copy(y_hbm.at[dst_v.at[c0 + 1]], rows1, sem1)
        pltpu.make_async_copy(y_hbm.at[dst_v.at[c0]], rows0, sem0).wait()
        pltpu.make_async_copy(src_hbm.at[wid, c0], srcb0, sem_s0).wait()
        pltpu.sync_copy(rows0, acc.at[srcb0.at[0]], add=True)
        pltpu.make_async_copy(y_hbm.at[dst_v.at[c0 + 1]], rows1, sem1).wait()
        pltpu.make_async_copy(src_hbm.at[wid, c0 + 1], srcb1, sem_s1).wait()
        pltpu.sync_copy(rows1, acc.at[srcb1.at[0]], add=True)
        plsc.subcore_barrier()

        pltpu.sync_copy(acc.at[pl.ds(sid * RPT, RPT)], out_hbm.at[cid, sid])

    return edge_pass


@functools.cache
def _make_deg_pass():
    mesh = plsc.VectorSubcoreMesh(
        core_axis_name="c", subcore_axis_name="s",
        num_cores=NC, num_subcores=NS)

    @functools.partial(
        pl.kernel,
        out_type=jax.ShapeDtypeStruct((NC, NS, RPT, H), jnp.float32),
        mesh=mesh,
        scratch_types=[
            pltpu.VMEM_SHARED((N, H), jnp.float32),
            pltpu.VMEM((NCHUNK, CHUNK), jnp.int32),
            pltpu.VMEM((CHUNK, H), jnp.float32),
        ],
    )
    def deg_pass(src_hbm, zeros_hbm, ones_hbm, deg_hbm,
                 degacc, src_v, ones_v):
        cid = lax.axis_index("c")
        sid = lax.axis_index("s")
        wid = cid * NS + sid

        pltpu.sync_copy(src_hbm.at[wid], src_v)
        pltpu.sync_copy(zeros_hbm.at[sid], degacc.at[pl.ds(sid * RPT, RPT)])
        pltpu.sync_copy(ones_hbm, ones_v)
        plsc.subcore_barrier()

        def body(c, carry):
            pltpu.sync_copy(ones_v, degacc.at[src_v.at[c]], add=True)
            return carry

        lax.fori_loop(0, NCHUNK, body, 0)
        plsc.subcore_barrier()
        pltpu.sync_copy(degacc.at[pl.ds(sid * RPT, RPT)], deg_hbm.at[cid, sid])

    return deg_pass


def _edge_pass(*args):
    return _make_edge_pass()(*args)


def _deg_pass(*args):
    return _make_deg_pass()(*args)


def _relu_mm_body(p_ref, d_ref, b_ref, w_ref, y_ref, dinv_ref):
    m = p_ref[0] + p_ref[1]
    deg = (d_ref[0] + d_ref[1])[:, 0:1]
    dinv = 1.0 / jnp.maximum(deg, 1.0)
    h = jnp.maximum(m * dinv + b_ref[...], 0.0)
    y_ref[...] = jnp.dot(h, w_ref[...], preferred_element_type=jnp.float32)
    dinv_ref[...] = jnp.broadcast_to(dinv, (BLK, 16))


def _relu_mm(p, d, b, w):
    return pl.pallas_call(
        _relu_mm_body,
        grid=(NBLK,),
        in_specs=[
            pl.BlockSpec((NC, BLK, H), lambda i: (0, i, 0)),
            pl.BlockSpec((NC, BLK, H), lambda i: (0, i, 0)),
            pl.BlockSpec((1, H), lambda i: (0, 0)),
            pl.BlockSpec((H, H), lambda i: (0, 0)),
        ],
        out_specs=[
            pl.BlockSpec((BLK, H), lambda i: (i, 0)),
            pl.BlockSpec((BLK, 16), lambda i: (i, 0)),
        ],
        out_shape=[
            jax.ShapeDtypeStruct((N, H), jnp.float32),
            jax.ShapeDtypeStruct((N, 16), jnp.float32),
        ],
    )(p, d, b, w)


def _final_body(p_ref, d_ref, b_ref, wp_ref, bp_ref, o_ref):
    m = p_ref[0] + p_ref[1]
    dinv = d_ref[:, 0:1]
    h = jnp.maximum(m * dinv + b_ref[...], 0.0)
    z = jnp.sum(h * wp_ref[...], axis=1, keepdims=True)
    r = jnp.abs(z + bp_ref[0:1, 0:1])
    o_ref[...] = jnp.broadcast_to(jnp.sum(r), (1, 1, 128))


def _final(p, d, b, wp, bp):
    return pl.pallas_call(
        _final_body,
        grid=(NBLK,),
        in_specs=[
            pl.BlockSpec((NC, BLK, H), lambda i: (0, i, 0)),
            pl.BlockSpec((BLK, 16), lambda i: (i, 0)),
            pl.BlockSpec((1, H), lambda i: (0, 0)),
            pl.BlockSpec((1, H), lambda i: (0, 0)),
            pl.BlockSpec((1, 128), lambda i: (0, 0)),
        ],
        out_specs=pl.BlockSpec((1, 1, 128), lambda i: (i, 0, 0)),
        out_shape=jax.ShapeDtypeStruct((G, 1, 128), jnp.float32),
    )(p, d, b, wp, bp)


def kernel(features, edge_index, lengths, emb_table, W1, b1, W2, b2, Wp, bp):
    del lengths
    f = features[0]
    src3d = edge_index[0].reshape(NW, NCHUNK, CHUNK)
    src4d = edge_index[0].reshape(NW, NCHUNK, 1, CHUNK)
    dst3d = edge_index[1].reshape(NW, NCHUNK, CHUNK)
    w1a = W1[:EMB]
    w1p = jnp.concatenate([jnp.zeros((1, H), W1.dtype), W1[EMB:]], axis=0)
    b1r = b1.reshape(1, H)
    b2r = b2.reshape(1, H)
    wpr = Wp.reshape(1, H)
    bpr = jnp.broadcast_to(bp.reshape(1, 1), (1, 128))
    zeros_init = jnp.zeros((NS, RPT, H), jnp.float32)
    ones_rows = jnp.ones((CHUNK, H), jnp.float32)

    y1 = _embed_mm(f, emb_table, w1a, w1p)
    d1 = _deg_pass(src3d, zeros_init, ones_rows).reshape(NC, N, H)
    p1 = _edge_pass(y1, src4d, dst3d, zeros_init).reshape(NC, N, H)
    y2, dinv = _relu_mm(p1, d1, b1r, W2)
    p2 = _edge_pass(y2, src4d, dst3d, zeros_init).reshape(NC, N, H)
    out = _final(p2, dinv, b2r, wpr, bpr)
    return out[:, 0, 0]


# speedup vs baseline: 1.2269x; 1.0030x over previous
import functools

import jax
import jax.numpy as jnp
from jax import lax
from jax.experimental import pallas as pl
from jax.experimental.pallas import tpu as pltpu
from jax.experimental.pallas import tpu_sc as plsc

N = 10000
F = 128
EMB = 32
H = 128
G = 10
E = 320000

NBLK = 10
BLK = N // NBLK

NC = 2
NS = 16
NW = NC * NS
CHUNK = 125
EPW = E // NW
NCHUNK = EPW // CHUNK
NPAIR = NCHUNK // 2
RPT = N // NS


def _embed_mm_body(f_ref, emb_ref, w1a_ref, w1p_ref, y_ref):
    f = f_ref[...]
    op = f[:, 0:1].astype(jnp.int32)
    cols = lax.broadcasted_iota(jnp.int32, (BLK, 128), 1)
    onehot = (op == cols).astype(jnp.float32)
    emb = jnp.dot(onehot, emb_ref[...], preferred_element_type=jnp.float32)
    y = jnp.dot(emb, w1a_ref[...], preferred_element_type=jnp.float32)
    y += jnp.dot(f, w1p_ref[...], preferred_element_type=jnp.float32)
    y_ref[...] = y


def _embed_mm(f, emb_table, w1a, w1p):
    return pl.pallas_call(
        _embed_mm_body,
        grid=(NBLK,),
        in_specs=[
            pl.BlockSpec((BLK, F), lambda i: (i, 0)),
            pl.BlockSpec((128, EMB), lambda i: (0, 0)),
            pl.BlockSpec((EMB, H), lambda i: (0, 0)),
            pl.BlockSpec((F, H), lambda i: (0, 0)),
        ],
        out_specs=pl.BlockSpec((BLK, H), lambda i: (i, 0)),
        out_shape=jax.ShapeDtypeStruct((N, H), jnp.float32),
    )(f, emb_table, w1a, w1p)


@functools.cache
def _make_edge_pass(with_deg):
    mesh = plsc.VectorSubcoreMesh(
        core_axis_name="c", subcore_axis_name="s",
        num_cores=NC, num_subcores=NS)

    out_type = [jax.ShapeDtypeStruct((NC, NS, RPT, H), jnp.float32)]
    if with_deg:
        out_type.append(jax.ShapeDtypeStruct((NC, NS, RPT, H), jnp.float32))

    @functools.partial(
        pl.kernel,
        out_type=out_type,
        mesh=mesh,
        scratch_types=[
            pltpu.VMEM_SHARED((N, H), jnp.float32),
            pltpu.VMEM((NCHUNK, CHUNK), jnp.int32),
            pltpu.VMEM((1, CHUNK), jnp.int32),
            pltpu.VMEM((1, CHUNK), jnp.int32),
            pltpu.VMEM((CHUNK, H), jnp.float32),
            pltpu.VMEM((CHUNK, H), jnp.float32),
            pltpu.SemaphoreType.DMA,
            pltpu.SemaphoreType.DMA,
            pltpu.SemaphoreType.DMA,
            pltpu.SemaphoreType.DMA,
        ],
    )
    def edge_pass(y_hbm, src_hbm, dst_hbm, zeros_hbm, ones_hbm, *refs):
        if with_deg:
            (out_hbm, deg_hbm, acc, dst_v, srcb0, srcb1, rows0, rows1,
             sem0, sem1, sem_s0, sem_s1) = refs
        else:
            (out_hbm, acc, dst_v, srcb0, srcb1, rows0, rows1,
             sem0, sem1, sem_s0, sem_s1) = refs
        cid = lax.axis_index("c")
        sid = lax.axis_index("s")
        wid = cid * NS + sid
        stripe = pl.ds(sid * RPT, RPT)

        pltpu.sync_copy(dst_hbm.at[wid], dst_v)
        pltpu.async_copy(src_hbm.at[wid, 0], srcb0, sem_s0)
        pltpu.async_copy(src_hbm.at[wid, 1], srcb1, sem_s1)
        pltpu.sync_copy(zeros_hbm.at[sid], acc.at[stripe])

        if with_deg:
            pltpu.sync_copy(ones_hbm, rows0)
            plsc.subcore_barrier()

            def deg_body(p, carry):
                c0 = 2 * p
                pltpu.make_async_copy(
                    src_hbm.at[wid, c0], srcb0, sem_s0).wait()
                pltpu.sync_copy(rows0, acc.at[srcb0.at[0]], add=True)
                pltpu.async_copy(src_hbm.at[wid, c0 + 2], srcb0, sem_s0)
                pltpu.make_async_copy(
                    src_hbm.at[wid, c0 + 1], srcb1, sem_s1).wait()
                pltpu.sync_copy(rows0, acc.at[srcb1.at[0]], add=True)
                pltpu.async_copy(src_hbm.at[wid, c0 + 3], srcb1, sem_s1)
                return carry

            lax.fori_loop(0, NPAIR - 1, deg_body, 0)
            c0 = NCHUNK - 2
            pltpu.make_async_copy(src_hbm.at[wid, c0], srcb0, sem_s0).wait()
            pltpu.sync_copy(rows0, acc.at[srcb0.at[0]], add=True)
            pltpu.make_async_copy(
                src_hbm.at[wid, c0 + 1], srcb1, sem_s1).wait()
            pltpu.sync_copy(rows0, acc.at[srcb1.at[0]], add=True)
            plsc.subcore_barrier()
            pltpu.sync_copy(acc.at[stripe], deg_hbm.at[cid, sid])
            pltpu.sync_copy(zeros_hbm.at[sid], acc.at[stripe])
            pltpu.async_copy(src_hbm.at[wid, 0], srcb0, sem_s0)
            pltpu.async_copy(src_hbm.at[wid, 1], srcb1, sem_s1)

        plsc.subcore_barrier()

        pltpu.async_copy(y_hbm.at[dst_v.at[0]], rows0, sem0)

        def body(p, carry):
            c0 = 2 * p
            pltpu.async_copy(y_hbm.at[dst_v.at[c0 + 1]], rows1, sem1)
            pltpu.make_async_copy(y_hbm.at[dst_v.at[c0]], rows0, sem0).wait()
            pltpu.make_async_copy(src_hbm.at[wid, c0], srcb0, sem_s0).wait()
            pltpu.sync_copy(rows0, acc.at[srcb0.at[0]], add=True)
            pltpu.async_copy(src_hbm.at[wid, c0 + 2], srcb0, sem_s0)
            pltpu.async_copy(y_hbm.at[dst_v.at[c0 + 2]], rows0, sem0)
            pltpu.make_async_copy(
                y_hbm.at[dst_v.at[c0 + 1]], rows1, sem1).wait()
            pltpu.make_async_copy(
                src_hbm.at[wid, c0 + 1], srcb1, sem_s1).wait()
            pltpu.sync_copy(rows1, acc.at[srcb1.at[0]], add=True)
            pltpu.async_copy(src_hbm.at[wid, c0 + 3], srcb1, sem_s1)
            return carry

        lax.fori_loop(0, NPAIR - 1, body, 0)
        c0 = NCHUNK - 2
        pltpu.async_copy(y_hbm.at[dst_v.at[c0 + 1]], rows1, sem1)
        pltpu.make_async_copy(y_hbm.at[dst_v.at[c0]], rows0, sem0).wait()
        pltpu.make_async_copy(src_hbm.at[wid, c0], srcb0, sem_s0).wait()
        pltpu.sync_copy(rows0, acc.at[srcb0.at[0]], add=True)
        pltpu.make_async_copy(y_hbm.at[dst_v.at[c0 + 1]], rows1, sem1).wait()
        pltpu.make_async_copy(src_hbm.at[wid, c0 + 1], srcb1, sem_s1).wait()
        pltpu.sync_copy(rows1, acc.at[srcb1.at[0]], add=True)
        plsc.subcore_barrier()

        pltpu.sync_copy(acc.at[stripe], out_hbm.at[cid, sid])

    return edge_pass


def _edge_pass(*args):
    (p,) = _make_edge_pass(False)(*args)
    return p


def _edge_pass_deg(*args):
    return _make_edge_pass(True)(*args)


def _relu_mm_body(p_ref, d_ref, b_ref, w_ref, y_ref, dinv_ref):
    m = p_ref[0] + p_ref[1]
    deg = (d_ref[0] + d_ref[1])[:, 0:1]
    dinv = 1.0 / jnp.maximum(deg, 1.0)
    h = jnp.maximum(m * dinv + b_ref[...], 0.0)
    y_ref[...] = jnp.dot(h, w_ref[...], preferred_element_type=jnp.float32)
    dinv_ref[...] = jnp.broadcast_to(dinv, (BLK, 16))


def _relu_mm(p, d, b, w):
    return pl.pallas_call(
        _relu_mm_body,
        grid=(NBLK,),
        in_specs=[
            pl.BlockSpec((NC, BLK, H), lambda i: (0, i, 0)),
            pl.BlockSpec((NC, BLK, H), lambda i: (0, i, 0)),
            pl.BlockSpec((1, H), lambda i: (0, 0)),
            pl.BlockSpec((H, H), lambda i: (0, 0)),
        ],
        out_specs=[
            pl.BlockSpec((BLK, H), lambda i: (i, 0)),
            pl.BlockSpec((BLK, 16), lambda i: (i, 0)),
        ],
        out_shape=[
            jax.ShapeDtypeStruct((N, H), jnp.float32),
            jax.ShapeDtypeStruct((N, 16), jnp.float32),
        ],
    )(p, d, b, w)


def _final_body(p_ref, d_ref, b_ref, wp_ref, bp_ref, o_ref):
    m = p_ref[0] + p_ref[1]
    dinv = d_ref[:, 0:1]
    h = jnp.maximum(m * dinv + b_ref[...], 0.0)
    z = jnp.sum(h * wp_ref[...], axis=1, keepdims=True)
    r = jnp.abs(z + bp_ref[0:1, 0:1])
    o_ref[...] = jnp.broadcast_to(jnp.sum(r), (1, 1, 128))


def _final(p, d, b, wp, bp):
    return pl.pallas_call(
        _final_body,
        grid=(NBLK,),
        in_specs=[
            pl.BlockSpec((NC, BLK, H), lambda i: (0, i, 0)),
            pl.BlockSpec((BLK, 16), lambda i: (i, 0)),
            pl.BlockSpec((1, H), lambda i: (0, 0)),
            pl.BlockSpec((1, H), lambda i: (0, 0)),
            pl.BlockSpec((1, 128), lambda i: (0, 0)),
        ],
        out_specs=pl.BlockSpec((1, 1, 128), lambda i: (i, 0, 0)),
        out_shape=jax.ShapeDtypeStruct((G, 1, 128), jnp.float32),
    )(p, d, b, wp, bp)


def kernel(features, edge_index, lengths, emb_table, W1, b1, W2, b2, Wp, bp):
    del lengths
    f = features[0]
    src4d = edge_index[0].reshape(NW, NCHUNK, 1, CHUNK)
    dst3d = edge_index[1].reshape(NW, NCHUNK, CHUNK)
    w1a = W1[:EMB]
    w1p = jnp.concatenate([jnp.zeros((1, H), W1.dtype), W1[EMB:]], axis=0)
    b1r = b1.reshape(1, H)
    b2r = b2.reshape(1, H)
    wpr = Wp.reshape(1, H)
    bpr = jnp.broadcast_to(bp.reshape(1, 1), (1, 128))
    zeros_init = jnp.zeros((NS, RPT, H), jnp.float32)
    ones_rows = jnp.ones((CHUNK, H), jnp.float32)

    y1 = _embed_mm(f, emb_table, w1a, w1p)
    p1, d1 = _edge_pass_deg(y1, src4d, dst3d, zeros_init, ones_rows)
    p1 = p1.reshape(NC, N, H)
    d1 = d1.reshape(NC, N, H)
    y2, dinv = _relu_mm(p1, d1, b1r, W2)
    p2 = _edge_pass(y2, src4d, dst3d, zeros_init, ones_rows)
    p2 = p2.reshape(NC, N, H)
    out = _final(p2, dinv, b2r, wpr, bpr)
    return out[:, 0, 0]


# speedup vs baseline: 1.3136x; 1.0707x over previous
import functools

import jax
import jax.numpy as jnp
from jax import lax
from jax.experimental import pallas as pl
from jax.experimental.pallas import tpu as pltpu
from jax.experimental.pallas import tpu_sc as plsc

N = 10000
F = 128
EMB = 32
H = 128
G = 10
E = 320000

NBLK = 10
BLK = N // NBLK

NC = 2
NS = 16
NW = NC * NS
CHUNK = 125
EPW = E // NW
NCHUNK = EPW // CHUNK
NPAIR = NCHUNK // 2
RPT = N // NS
STRIPE = 624
REM = N - NS * STRIPE


def _embed_mm_body(f_ref, emb_ref, w1a_ref, w1p_ref, y_ref):
    f = f_ref[...]
    op = f[:, 0:1].astype(jnp.int32)
    cols = lax.broadcasted_iota(jnp.int32, (BLK, 128), 1)
    onehot = (op == cols).astype(jnp.float32)
    emb = jnp.dot(onehot, emb_ref[...], preferred_element_type=jnp.float32)
    y = jnp.dot(emb, w1a_ref[...], preferred_element_type=jnp.float32)
    y += jnp.dot(f, w1p_ref[...], preferred_element_type=jnp.float32)
    y_ref[...] = y


def _embed_mm(f, emb_table, w1a, w1p):
    return pl.pallas_call(
        _embed_mm_body,
        grid=(NBLK,),
        in_specs=[
            pl.BlockSpec((BLK, F), lambda i: (i, 0)),
            pl.BlockSpec((128, EMB), lambda i: (0, 0)),
            pl.BlockSpec((EMB, H), lambda i: (0, 0)),
            pl.BlockSpec((F, H), lambda i: (0, 0)),
        ],
        out_specs=pl.BlockSpec((BLK, H), lambda i: (i, 0)),
        out_shape=jax.ShapeDtypeStruct((N, H), jnp.float32),
    )(f, emb_table, w1a, w1p)


@functools.cache
def _make_edge_pass(with_deg):
    mesh = plsc.VectorSubcoreMesh(
        core_axis_name="c", subcore_axis_name="s",
        num_cores=NC, num_subcores=NS)

    out_type = [jax.ShapeDtypeStruct((NC, N, H), jnp.float32)]
    if with_deg:
        out_type.append(jax.ShapeDtypeStruct((NC, N, H), jnp.float32))

    @functools.partial(
        pl.kernel,
        out_type=out_type,
        mesh=mesh,
        scratch_types=[
            pltpu.VMEM_SHARED((N, H), jnp.float32),
            pltpu.VMEM((NCHUNK, CHUNK), jnp.int32),
            pltpu.VMEM((1, CHUNK), jnp.int32),
            pltpu.VMEM((1, CHUNK), jnp.int32),
            pltpu.VMEM((CHUNK, H), jnp.float32),
            pltpu.VMEM((CHUNK, H), jnp.float32),
            pltpu.SemaphoreType.DMA,
            pltpu.SemaphoreType.DMA,
            pltpu.SemaphoreType.DMA,
            pltpu.SemaphoreType.DMA,
        ],
    )
    def edge_pass(y_hbm, src_hbm, dst_hbm, zeros_hbm, ones_hbm, *refs):
        if with_deg:
            (out_hbm, deg_hbm, acc, dst_v, srcb0, srcb1, rows0, rows1,
             sem0, sem1, sem_s0, sem_s1) = refs
        else:
            (out_hbm, acc, dst_v, srcb0, srcb1, rows0, rows1,
             sem0, sem1, sem_s0, sem_s1) = refs
        cid = lax.axis_index("c")
        sid = lax.axis_index("s")
        wid = cid * NS + sid
        stripe = pl.ds(sid * RPT, RPT)

        def drain(target_hbm):
            dstripe = pl.ds(sid * STRIPE, STRIPE)
            pltpu.sync_copy(acc.at[dstripe], target_hbm.at[cid, dstripe])

            @pl.when(sid == 0)
            def _():
                tail = pl.ds(NS * STRIPE, REM)
                pltpu.sync_copy(acc.at[tail], target_hbm.at[cid, tail])

        pltpu.sync_copy(dst_hbm.at[wid], dst_v)
        pltpu.async_copy(src_hbm.at[wid, 0], srcb0, sem_s0)
        pltpu.async_copy(src_hbm.at[wid, 1], srcb1, sem_s1)
        pltpu.sync_copy(zeros_hbm.at[sid], acc.at[stripe])

        if with_deg:
            pltpu.sync_copy(ones_hbm, rows0)
            plsc.subcore_barrier()

            def deg_body(p, carry):
                c0 = 2 * p
                pltpu.make_async_copy(
                    src_hbm.at[wid, c0], srcb0, sem_s0).wait()
                pltpu.sync_copy(rows0, acc.at[srcb0.at[0]], add=True)
                pltpu.async_copy(src_hbm.at[wid, c0 + 2], srcb0, sem_s0)
                pltpu.make_async_copy(
                    src_hbm.at[wid, c0 + 1], srcb1, sem_s1).wait()
                pltpu.sync_copy(rows0, acc.at[srcb1.at[0]], add=True)
                pltpu.async_copy(src_hbm.at[wid, c0 + 3], srcb1, sem_s1)
                return carry

            lax.fori_loop(0, NPAIR - 1, deg_body, 0)
            c0 = NCHUNK - 2
            pltpu.make_async_copy(src_hbm.at[wid, c0], srcb0, sem_s0).wait()
            pltpu.sync_copy(rows0, acc.at[srcb0.at[0]], add=True)
            pltpu.make_async_copy(
                src_hbm.at[wid, c0 + 1], srcb1, sem_s1).wait()
            pltpu.sync_copy(rows0, acc.at[srcb1.at[0]], add=True)
            plsc.subcore_barrier()
            drain(deg_hbm)
            pltpu.sync_copy(zeros_hbm.at[sid], acc.at[stripe])
            pltpu.async_copy(src_hbm.at[wid, 0], srcb0, sem_s0)
            pltpu.async_copy(src_hbm.at[wid, 1], srcb1, sem_s1)

        plsc.subcore_barrier()

        pltpu.async_copy(y_hbm.at[dst_v.at[0]], rows0, sem0)

        def body(p, carry):
            c0 = 2 * p
            pltpu.async_copy(y_hbm.at[dst_v.at[c0 + 1]], rows1, sem1)
            pltpu.make_async_copy(y_hbm.at[dst_v.at[c0]], rows0, sem0).wait()
            pltpu.make_async_copy(src_hbm.at[wid, c0], srcb0, sem_s0).wait()
            pltpu.sync_copy(rows0, acc.at[srcb0.at[0]], add=True)
            pltpu.async_copy(src_hbm.at[wid, c0 + 2], srcb0, sem_s0)
            pltpu.async_copy(y_hbm.at[dst_v.at[c0 + 2]], rows0, sem0)
            pltpu.make_async_copy(
                y_hbm.at[dst_v.at[c0 + 1]], rows1, sem1).wait()
            pltpu.make_async_copy(
                src_hbm.at[wid, c0 + 1], srcb1, sem_s1).wait()
            pltpu.sync_copy(rows1, acc.at[srcb1.at[0]], add=True)
            pltpu.async_copy(src_hbm.at[wid, c0 + 3], srcb1, sem_s1)
            return carry

        lax.fori_loop(0, NPAIR - 1, body, 0)
        c0 = NCHUNK - 2
        pltpu.async_copy(y_hbm.at[dst_v.at[c0 + 1]], rows1, sem1)
        pltpu.make_async_copy(y_hbm.at[dst_v.at[c0]], rows0, sem0).wait()
        pltpu.make_async_copy(src_hbm.at[wid, c0], srcb0, sem_s0).wait()
        pltpu.sync_copy(rows0, acc.at[srcb0.at[0]], add=True)
        pltpu.make_async_copy(y_hbm.at[dst_v.at[c0 + 1]], rows1, sem1).wait()
        pltpu.make_async_copy(src_hbm.at[wid, c0 + 1], srcb1, sem_s1).wait()
        pltpu.sync_copy(rows1, acc.at[srcb1.at[0]], add=True)
        plsc.subcore_barrier()

        drain(out_hbm)

    return edge_pass


def _edge_pass(*args):
    (p,) = _make_edge_pass(False)(*args)
    return p


def _edge_pass_deg(*args):
    return _make_edge_pass(True)(*args)


def _relu_mm_body(p_ref, d_ref, b_ref, w_ref, y_ref, dinv_ref):
    m = p_ref[0] + p_ref[1]
    deg = (d_ref[0] + d_ref[1])[:, 0:1]
    dinv = 1.0 / jnp.maximum(deg, 1.0)
    h = jnp.maximum(m * dinv + b_ref[...], 0.0)
    y_ref[...] = jnp.dot(h, w_ref[...], preferred_element_type=jnp.float32)
    dinv_ref[...] = jnp.broadcast_to(dinv, (BLK, 16))


def _relu_mm(p, d, b, w):
    return pl.pallas_call(
        _relu_mm_body,
        grid=(NBLK,),
        in_specs=[
            pl.BlockSpec((NC, BLK, H), lambda i: (0, i, 0)),
            pl.BlockSpec((NC, BLK, H), lambda i: (0, i, 0)),
            pl.BlockSpec((1, H), lambda i: (0, 0)),
            pl.BlockSpec((H, H), lambda i: (0, 0)),
        ],
        out_specs=[
            pl.BlockSpec((BLK, H), lambda i: (i, 0)),
            pl.BlockSpec((BLK, 16), lambda i: (i, 0)),
        ],
        out_shape=[
            jax.ShapeDtypeStruct((N, H), jnp.float32),
            jax.ShapeDtypeStruct((N, 16), jnp.float32),
        ],
    )(p, d, b, w)


def _final_body(p_ref, d_ref, b_ref, wp_ref, bp_ref, o_ref):
    m = p_ref[0] + p_ref[1]
    dinv = d_ref[:, 0:1]
    h = jnp.maximum(m * dinv + b_ref[...], 0.0)
    z = jnp.sum(h * wp_ref[...], axis=1, keepdims=True)
    r = jnp.abs(z + bp_ref[0:1, 0:1])
    o_ref[...] = jnp.broadcast_to(jnp.sum(r), (1, 1, 128))


def _final(p, d, b, wp, bp):
    return pl.pallas_call(
        _final_body,
        grid=(NBLK,),
        in_specs=[
            pl.BlockSpec((NC, BLK, H), lambda i: (0, i, 0)),
            pl.BlockSpec((BLK, 16), lambda i: (i, 0)),
            pl.BlockSpec((1, H), lambda i: (0, 0)),
            pl.BlockSpec((1, H), lambda i: (0, 0)),
            pl.BlockSpec((1, 128), lambda i: (0, 0)),
        ],
        out_specs=pl.BlockSpec((1, 1, 128), lambda i: (i, 0, 0)),
        out_shape=jax.ShapeDtypeStruct((G, 1, 128), jnp.float32),
    )(p, d, b, wp, bp)


def kernel(features, edge_index, lengths, emb_table, W1, b1, W2, b2, Wp, bp):
    del lengths
    f = features[0]
    src4d = edge_index[0].reshape(NW, NCHUNK, 1, CHUNK)
    dst3d = edge_index[1].reshape(NW, NCHUNK, CHUNK)
    w1a = W1[:EMB]
    w1p = jnp.concatenate([jnp.zeros((1, H), W1.dtype), W1[EMB:]], axis=0)
    b1r = b1.reshape(1, H)
    b2r = b2.reshape(1, H)
    wpr = Wp.reshape(1, H)
    bpr = jnp.broadcast_to(bp.reshape(1, 1), (1, 128))
    zeros_init = jnp.zeros((NS, RPT, H), jnp.float32)
    ones_rows = jnp.ones((CHUNK, H), jnp.float32)

    y1 = _embed_mm(f, emb_table, w1a, w1p)
    p1, d1 = _edge_pass_deg(y1, src4d, dst3d, zeros_init, ones_rows)
    y2, dinv = _relu_mm(p1, d1, b1r, W2)
    p2 = _edge_pass(y2, src4d, dst3d, zeros_init, ones_rows)
    out = _final(p2, dinv, b2r, wpr, bpr)
    return out[:, 0, 0]
